# Initial kernel scaffold; baseline (speedup 1.0000x reference)
import jax
import jax.numpy as jnp
from jax.experimental import pallas as pl


def kernel(x, batch_assignment, W, b):
    raise NotImplementedError("write your pallas kernel here")


# speedup vs baseline: 4.1475x; 4.1475x over previous
import jax
import jax.numpy as jnp
from jax import lax
from jax.experimental import pallas as pl
from jax.experimental.pallas import tpu as pltpu
from jax.experimental.pallas import tpu_sc as plsc

N = 100000
H = 256
G = 256
NC = 2
NS = 16
NW = NC * NS
CHUNK = 80
NCHK = N // CHUNK
JMAX = -(-NCHK // NW)
CW = 16
L = 16
CHUNK_P = 128
NG = H // L


def _sc_segment_sum(x3, ids3, zsheet, zcnt):
    mesh = plsc.VectorSubcoreMesh(
        core_axis_name="c", subcore_axis_name="s",
        num_cores=NC, num_subcores=NS)

    def body(x_hbm, ids_hbm, zsheet_hbm, zcnt_hbm,
             s_out, c_out,
             idx_v, buf_v, s_acc, c_acc):
        cid = lax.axis_index("c")
        sid = lax.axis_index("s")
        wid = sid * NC + cid

        pltpu.sync_copy(zsheet_hbm, s_acc)
        pltpu.sync_copy(zcnt_hbm, c_acc)
        pltpu.sync_copy(ids_hbm.at[wid], idx_v)

        def do_row(st, seg, r):
            cur, run, accs = st
            changed = seg != cur
            new_accs = []
            for c in range(NG):
                v = buf_v[r, pl.ds(c * L, L)]
                new_accs.append(jnp.where(changed, v, accs[c] + v))
            run = jnp.where(changed, jnp.float32(1.0), run + 1.0)
            for c in range(NG):
                s_acc[seg, pl.ds(c * L, L)] = new_accs[c]
            c_acc[seg, :] = jnp.broadcast_to(run, (CW,))
            return (seg, run, tuple(new_accs))

        def chunk_step(j, st):
            c = wid + j * NW
            pltpu.sync_copy(x_hbm.at[pl.ds(c * CHUNK, CHUNK)], buf_v)

            def group_step(rb, st2):
                segvec = idx_v[j, pl.ds(rb * L, L)]
                for rr in range(L):
                    st2 = do_row(st2, segvec[rr], rb * L + rr)
                return st2

            return lax.fori_loop(0, CHUNK // L, group_step, st)

        st0 = (jnp.int32(-1), jnp.float32(0.0),
               tuple(jnp.zeros((L,), jnp.float32) for _ in range(NG)))
        nch = jnp.where(wid < NCHK - (JMAX - 1) * NW, JMAX, JMAX - 1)
        lax.fori_loop(0, nch, chunk_step, st0)

        pltpu.sync_copy(s_acc, s_out.at[wid])
        pltpu.sync_copy(c_acc, c_out.at[wid])

    call = pl.kernel(
        body,
        out_type=(
            jax.ShapeDtypeStruct((NW, G, H), jnp.float32),
            jax.ShapeDtypeStruct((NW, G, CW), jnp.float32),
        ),
        mesh=mesh,
        scratch_types=[
            pltpu.VMEM((JMAX, CHUNK_P), jnp.int32),
            pltpu.VMEM((CHUNK, H), jnp.float32),
            pltpu.VMEM((G, H), jnp.float32),
            pltpu.VMEM((G, CW), jnp.float32),
        ],
    )
    return call(x3, ids3, zsheet, zcnt)


def _tc_finish_body(s_ref, c_ref, w_ref, b_ref, o_ref):
    S = jnp.sum(s_ref[...], axis=0)
    cnt = jnp.sum(c_ref[...], axis=0)[:, 0:1]
    M = lax.dot_general(S, w_ref[...],
                        dimension_numbers=(((1,), (1,)), ((), ())),
                        preferred_element_type=jnp.float32)
    pos = cnt > 0.0
    inv = jnp.where(pos, 1.0 / jnp.where(pos, cnt, 1.0), 0.0)
    o_ref[...] = M * inv + jnp.where(pos, b_ref[...], 0.0)


@jax.jit
def kernel(x, batch_assignment, W, b):
    ids = batch_assignment.astype(jnp.int32)
    x3 = x
    idsp = jnp.pad(ids, (0, JMAX * NW * CHUNK - N))
    ids3 = jnp.pad(
        idsp.reshape(JMAX, NW, CHUNK).transpose(1, 0, 2),
        ((0, 0), (0, 0), (0, CHUNK_P - CHUNK)))
    zsheet = jnp.zeros((G, H), jnp.float32)
    zcnt = jnp.zeros((G, CW), jnp.float32)

    s_parts, c_parts = _sc_segment_sum(x3, ids3, zsheet, zcnt)

    out = pl.pallas_call(
        _tc_finish_body,
        out_shape=jax.ShapeDtypeStruct((G, H), jnp.float32),
    )(s_parts, c_parts, W, b.reshape(1, H))
    return out
